# Initial kernel scaffold; baseline (speedup 1.0000x reference)
import jax
import jax.numpy as jnp
from jax.experimental import pallas as pl


def kernel(x, obj_table, color_table, state_table, pos_table, W1, b1, g1, be1, W2, b2, g2, be2):
    raise NotImplementedError("write your pallas kernel here")


# speedup vs baseline: 10.2301x; 10.2301x over previous
import functools

import jax
import jax.numpy as jnp
from jax import lax
from jax.experimental import pallas as pl
from jax.experimental.pallas import tpu as pltpu
from jax.experimental.pallas import tpu_sc as plsc

B, H, W, ED, D = 4096, 7, 7, 16, 64
P = H * W
NCOMBO = 11 * 6 * 3 * P
NLUT = 9728
TOK = B * P

NW = 32
TPW = TOK // NW
CHUNK = 128
NCH = TPW // CHUNK
NBUF = 4


def _tc_body(x2_ref, m_ref, tblk_ref, w1_ref, b1_ref, g1_ref, be1_ref,
             w2_ref, b2_ref, g2_ref, be2_ref, lut_ref, idx_ref):
    acc = jnp.dot(x2_ref[...], m_ref[...], preferred_element_type=jnp.float32, precision=lax.Precision.HIGHEST)
    posi = lax.broadcasted_iota(jnp.int32, (B, P), 1)
    idx_ref[...] = acc.astype(jnp.int32) + posi

    k = lax.broadcasted_iota(jnp.int32, (NLUT, 128), 0)
    col = lax.broadcasted_iota(jnp.int32, (NLUT, 128), 1)
    p = k % P
    rem = k // P
    o = rem % 11
    c = (rem // 11) % 6
    s = rem // 66
    oh = ((col == o) & (col < 11)) \
        | ((col - 32 == c) & (col >= 32) & (col < 38)) \
        | ((col - 64 == s) & (col >= 64) & (col < 67)) \
        | (col - 79 == p)
    e = jnp.dot(oh.astype(jnp.float32), tblk_ref[...],
                preferred_element_type=jnp.float32, precision=lax.Precision.HIGHEST)

    h = jnp.dot(e, w1_ref[...], preferred_element_type=jnp.float32, precision=lax.Precision.HIGHEST) + b1_ref[...]
    a, g = h[:, :D], h[:, D:]
    h = a * jax.nn.sigmoid(g)
    mu = jnp.mean(h, axis=-1, keepdims=True)
    var = jnp.mean((h - mu) ** 2, axis=-1, keepdims=True)
    h = (h - mu) * lax.rsqrt(var + 1e-5) * g1_ref[...] + be1_ref[...]

    h = jnp.dot(h, w2_ref[...], preferred_element_type=jnp.float32, precision=lax.Precision.HIGHEST) + b2_ref[...]
    a, g = h[:, :D], h[:, D:]
    h = a * jax.nn.sigmoid(g)
    mu = jnp.mean(h, axis=-1, keepdims=True)
    var = jnp.mean((h - mu) ** 2, axis=-1, keepdims=True)
    lut_ref[...] = (h - mu) * lax.rsqrt(var + 1e-5) * g2_ref[...] + be2_ref[...]


def _sc_gather(lut_hbm, idx_hbm, out_hbm, idx_v, rows_v, sems):
    wid = lax.axis_index("s") * 2 + lax.axis_index("c")
    base = wid * TPW
    pltpu.sync_copy(idx_hbm.at[wid], idx_v)

    def fire(j, slot):
        pltpu.async_copy(lut_hbm.at[idx_v.at[j]], rows_v.at[slot], sems.at[slot])

    for j in range(NBUF - 1):
        fire(j, j)

    def body(j, _):
        slot = lax.rem(j, NBUF)
        pltpu.make_async_copy(lut_hbm.at[idx_v.at[j]], rows_v.at[slot],
                              sems.at[slot]).wait()
        pltpu.sync_copy(rows_v.at[slot], out_hbm.at[pl.ds(base + j * CHUNK, CHUNK)])
        nxt = j + (NBUF - 1)

        @pl.when(nxt < NCH)
        def _():
            fire(nxt, lax.rem(nxt, NBUF))
        return _

    lax.fori_loop(0, NCH, body, None)


def kernel(x, obj_table, color_table, state_table, pos_table,
           W1, b1, g1, be1, W2, b2, g2, be2):
    x2 = x.reshape(B, P * 3).astype(jnp.float32)

    m = jnp.zeros((P * 3, P), jnp.float32)
    jj = jnp.arange(P)
    m = m.at[3 * jj + 0, jj].set(49.0)
    m = m.at[3 * jj + 1, jj].set(539.0)
    m = m.at[3 * jj + 2, jj].set(3234.0)

    tblk = jnp.zeros((128, 4 * ED), jnp.float32)
    tblk = tblk.at[0:11, 0:ED].set(obj_table)
    tblk = tblk.at[32:38, ED:2 * ED].set(color_table)
    tblk = tblk.at[64:67, 2 * ED:3 * ED].set(state_table)
    tblk = tblk.at[79:128, 3 * ED:4 * ED].set(pos_table)

    lut, idx = pl.pallas_call(
        _tc_body,
        out_shape=[
            jax.ShapeDtypeStruct((NLUT, D), jnp.float32),
            jax.ShapeDtypeStruct((B, P), jnp.int32),
        ],
    )(x2, m, tblk, W1, b1.reshape(1, 2 * D), g1.reshape(1, D),
      be1.reshape(1, D), W2, b2.reshape(1, 2 * D), g2.reshape(1, D),
      be2.reshape(1, D))

    idx3 = idx.reshape(NW, NCH, CHUNK)

    mesh = plsc.VectorSubcoreMesh(core_axis_name="c", subcore_axis_name="s")
    gather = functools.partial(
        pl.kernel,
        mesh=mesh,
        compiler_params=pltpu.CompilerParams(use_tc_tiling_on_sc=False),
        out_type=jax.ShapeDtypeStruct((TOK, D), jnp.float32),
        scratch_types=[
            pltpu.VMEM((NCH, CHUNK), jnp.int32),
            pltpu.VMEM((NBUF, CHUNK, D), jnp.float32),
            pltpu.SemaphoreType.DMA((NBUF,)),
        ],
    )(_sc_gather)

    out = gather(lut, idx3)
    return out.reshape(B, H, W, D)
